# baseline (speedup 1.0000x reference)
import functools

import jax
import jax.numpy as jnp
from jax import lax
from jax.experimental import pallas as pl
from jax.experimental.pallas import tpu as pltpu
from jax.experimental.pallas import tpu_sc as plsc

D_MODEL = 64
SCALE = 8.0
B_TOTAL = 4096 * 200
NUM_WORKERS = 32
PER_W = B_TOTAL // NUM_WORKERS
CHUNK = 512
N_CHUNKS = PER_W // CHUNK
IDX_W = 128
GPC = CHUNK // IDX_W
IDX_ROWS_PER_W = PER_W // IDX_W

_mesh = plsc.VectorSubcoreMesh(core_axis_name="c", subcore_axis_name="s")


@functools.partial(
    pl.kernel,
    mesh=_mesh,
    out_type=jax.ShapeDtypeStruct((B_TOTAL, D_MODEL), jnp.float32),
    compiler_params=pltpu.CompilerParams(use_tc_tiling_on_sc=False),
    scratch_types=[
        pltpu.VMEM((GPC, IDX_W), jnp.int32),
        pltpu.VMEM((GPC, IDX_W), jnp.int32),
        pltpu.VMEM((CHUNK, D_MODEL), jnp.float32),
        pltpu.VMEM((CHUNK, D_MODEL), jnp.float32),
        pltpu.SemaphoreType.DMA,
        pltpu.SemaphoreType.DMA,
        pltpu.SemaphoreType.DMA,
        pltpu.SemaphoreType.DMA,
    ],
)
def _emb_lookup(idx_hbm, lut_hbm, out_hbm, ib0, ib1, rb0, rb1,
                gs0, gs1, os0, os1):
    wid = lax.axis_index("s") * 2 + lax.axis_index("c")
    ibufs = (ib0, ib1)
    rbufs = (rb0, rb1)
    gsems = (gs0, gs1)
    osems = (os0, os1)

    def load_idx(g, b):
        row = wid * IDX_ROWS_PER_W + g * GPC
        pltpu.sync_copy(idx_hbm.at[pl.ds(row, GPC)], ibufs[b])

    def fire_gathers(b):
        for j in range(GPC):
            pltpu.async_copy(
                lut_hbm.at[ibufs[b].at[j]],
                rbufs[b].at[pl.ds(j * IDX_W, IDX_W)],
                gsems[b],
            )

    def wait_gathers(b):
        for j in range(GPC):
            pltpu.make_async_copy(
                lut_hbm.at[ibufs[b].at[j]],
                rbufs[b].at[pl.ds(j * IDX_W, IDX_W)],
                gsems[b],
            ).wait()

    def scale(b):
        rb = rbufs[b]

        def body(i, carry):
            for j in range(D_MODEL // 16):
                rb[i, pl.ds(j * 16, 16)] = rb[i, pl.ds(j * 16, 16)] * SCALE
            return carry

        lax.fori_loop(0, CHUNK, body, 0)

    def fire_out(g, b):
        base = wid * PER_W + g * CHUNK
        pltpu.async_copy(rbufs[b], out_hbm.at[pl.ds(base, CHUNK)], osems[b])

    def wait_out(g, b):
        base = wid * PER_W + g * CHUNK
        pltpu.make_async_copy(
            rbufs[b], out_hbm.at[pl.ds(base, CHUNK)], osems[b]
        ).wait()

    load_idx(0, 0)
    fire_gathers(0)
    for g in range(N_CHUNKS):
        b = g & 1
        nb = 1 - b
        if g + 1 < N_CHUNKS:
            load_idx(g + 1, nb)
            if g >= 1:
                wait_out(g - 1, nb)
            fire_gathers(nb)
        wait_gathers(b)
        scale(b)
        fire_out(g, b)
    wait_out(N_CHUNKS - 2, (N_CHUNKS - 2) & 1)
    wait_out(N_CHUNKS - 1, (N_CHUNKS - 1) & 1)


def kernel(x, lut):
    xf = x.reshape(B_TOTAL // IDX_W, IDX_W).astype(jnp.int32)
    out = _emb_lookup(xf, lut)
    return out.reshape(x.shape[0], x.shape[1], D_MODEL)
